# Initial kernel scaffold; baseline (speedup 1.0000x reference)
import jax
import jax.numpy as jnp
from jax.experimental import pallas as pl


def kernel(p_locs, p_clss, bboxes, labels, priors):
    raise NotImplementedError("write your pallas kernel here")


# speedup vs baseline: 8.6376x; 8.6376x over previous
import jax
import jax.numpy as jnp
from jax.experimental import pallas as pl

B, N, NOBJ, NCLS = 16, 24564, 16, 81
THR = 0.5
NEG_RATIO = 3.0


def _match_kernel(bboxes_ref, labels_ref, priors_t_ref, plocs_t_ref,
                  label_out_ref, scal_out_ref):
    pcx = priors_t_ref[0:1, :]
    pcy = priors_t_ref[1:2, :]
    pw = priors_t_ref[2:3, :]
    ph = priors_t_ref[3:4, :]
    px0 = pcx - pw / 2.0
    py0 = pcy - ph / 2.0
    px1 = pcx + pw / 2.0
    py1 = pcy + ph / 2.0

    bb = bboxes_ref[0]
    bx0 = bb[:, 0:1]
    by0 = bb[:, 1:2]
    bx1 = bb[:, 2:3]
    by1 = bb[:, 3:4]

    iw = jnp.maximum(jnp.minimum(bx1, px1) - jnp.maximum(bx0, px0), 0.0)
    ih = jnp.maximum(jnp.minimum(by1, py1) - jnp.maximum(by0, py0), 0.0)
    inter = iw * ih
    area_a = (bx1 - bx0) * (by1 - by0)
    area_b = (px1 - px0) * (py1 - py0)
    sc = inter / (area_a + area_b - inter)

    obj_iota = jax.lax.broadcasted_iota(jnp.int32, (NOBJ, N), 0)
    lane_iota = jax.lax.broadcasted_iota(jnp.int32, (NOBJ, N), 1)

    best_iou = jnp.max(sc, axis=0, keepdims=True)
    pbb = jnp.min(jnp.where(sc == best_iou, obj_iota, NOBJ), axis=0,
                  keepdims=True)
    bbp_val = jnp.max(sc, axis=1, keepdims=True)
    bbp = jnp.min(jnp.where(sc == bbp_val, lane_iota, N), axis=1,
                  keepdims=True)

    eq = lane_iota == bbp
    fj = jnp.max(jnp.where(eq, obj_iota, -1), axis=0, keepdims=True)
    forced = fj >= 0
    pbb = jnp.where(forced, fj, pbb)
    best_iou = jnp.where(forced, 1.0, best_iou)

    lbl = jnp.zeros((1, N), jnp.int32)
    gx0 = jnp.zeros((1, N), jnp.float32)
    gy0 = jnp.zeros((1, N), jnp.float32)
    gx1 = jnp.zeros((1, N), jnp.float32)
    gy1 = jnp.zeros((1, N), jnp.float32)
    for j in range(NOBJ):
        sel = pbb == j
        lbl = jnp.where(sel, labels_ref[0, 0, j], lbl)
        gx0 = jnp.where(sel, bb[j, 0], gx0)
        gy0 = jnp.where(sel, bb[j, 1], gy0)
        gx1 = jnp.where(sel, bb[j, 2], gx1)
        gy1 = jnp.where(sel, bb[j, 3], gy1)
    lbl = jnp.where(best_iou < THR, 0, lbl)
    pos = lbl != 0

    cx = (gx0 + gx1) / 2.0
    cy = (gy0 + gy1) / 2.0
    w = gx1 - gx0
    h = gy1 - gy0
    tl = jnp.concatenate(
        [(cx - pcx) / (pw / 10.0),
         (cy - pcy) / (ph / 10.0),
         jnp.log(w / pw) * 5.0,
         jnp.log(h / ph) * 5.0], axis=0)

    pl4 = plocs_t_ref[0]
    loc_l1 = jnp.sum(jnp.where(pos, jnp.abs(pl4 - tl), 0.0))
    n_pos = jnp.sum(jnp.where(pos, 1.0, 0.0))

    label_out_ref[:] = lbl.reshape(1, 1, N)
    io = jax.lax.broadcasted_iota(jnp.int32, (1, 128), 1)
    scal = jnp.where(io == 0, n_pos, jnp.where(io == 1, loc_l1, 0.0))
    scal_out_ref[:] = scal.reshape(1, 1, 128)


PCH = 4096
NCH = (N + PCH - 1) // PCH


def _ce_kernel(pclss_ref, label_ref, ceneg_out_ref, scal_out_ref):
    c = pl.program_id(1)
    x = pclss_ref[0]
    lbl = label_ref[0]
    m = jnp.max(x, axis=1, keepdims=True)
    s = jnp.sum(jnp.exp(x - m), axis=1, keepdims=True)
    lse = m + jnp.log(s)
    cls_iota = jax.lax.broadcasted_iota(jnp.int32, (PCH, NCLS), 1)
    picked = jnp.sum(jnp.where(cls_iota == lbl, x, 0.0), axis=1,
                     keepdims=True)
    ce = lse - picked
    posm = lbl != 0
    ceneg_out_ref[:] = jnp.where(posm, 0.0, ce).reshape(1, PCH, 1)

    row = jax.lax.broadcasted_iota(jnp.int32, (PCH, 1), 0) + c * PCH
    valid = jnp.logical_and(row < N, posm)
    conf_pos = jnp.sum(jnp.where(valid, ce, 0.0))

    @pl.when(c == 0)
    def _():
        scal_out_ref[:] = jnp.zeros((1, 1, 128), jnp.float32)

    io = jax.lax.broadcasted_iota(jnp.int32, (1, 128), 1)
    scal_out_ref[:] = scal_out_ref[:] + jnp.where(
        io == 0, conf_pos, 0.0).reshape(1, 1, 128)


def _reduce_kernel(ceneg_ref, scal1_ref, scal2_ref, out_ref):
    v = ceneg_ref[:]
    n_pos = scal1_ref[:, 0, 0:1]
    loc_l1 = scal1_ref[:, 0, 1:2]
    conf_pos = scal2_ref[:, 0, 0:1]
    k = jnp.minimum(n_pos * NEG_RATIO, float(N))

    prefix = jnp.zeros((B, 1), jnp.int32)
    for bit in range(30, -1, -1):
        cand = prefix | (1 << bit)
        candf = jax.lax.bitcast_convert_type(cand, jnp.float32)
        cnt = jnp.sum(jnp.where(v >= candf, 1.0, 0.0), axis=1, keepdims=True)
        prefix = jnp.where(cnt >= k, cand, prefix)
    t = jax.lax.bitcast_convert_type(prefix, jnp.float32)

    gt = v > t
    cnt_gt = jnp.sum(jnp.where(gt, 1.0, 0.0), axis=1, keepdims=True)
    sum_gt = jnp.sum(jnp.where(gt, v, 0.0), axis=1, keepdims=True)
    hard = sum_gt + (k - cnt_gt) * t

    n_total = jnp.sum(n_pos)
    conf = (jnp.sum(hard) + jnp.sum(conf_pos)) / n_total
    loc = jnp.sum(loc_l1) / (n_total * 4.0)
    out_ref[:] = jnp.full((8, 128), conf + loc, jnp.float32)


@jax.jit
def kernel(p_locs, p_clss, bboxes, labels, priors):
    priors_t = priors.T
    plocs_t = p_locs.transpose(0, 2, 1)
    labels3 = labels.reshape(B, 1, NOBJ)

    lab_out, scal1 = pl.pallas_call(
        _match_kernel,
        grid=(B,),
        in_specs=[
            pl.BlockSpec((1, NOBJ, 4), lambda b: (b, 0, 0)),
            pl.BlockSpec((1, 1, NOBJ), lambda b: (b, 0, 0)),
            pl.BlockSpec((4, N), lambda b: (0, 0)),
            pl.BlockSpec((1, 4, N), lambda b: (b, 0, 0)),
        ],
        out_specs=[
            pl.BlockSpec((1, 1, N), lambda b: (b, 0, 0)),
            pl.BlockSpec((1, 1, 128), lambda b: (b, 0, 0)),
        ],
        out_shape=[
            jax.ShapeDtypeStruct((B, 1, N), jnp.int32),
            jax.ShapeDtypeStruct((B, 1, 128), jnp.float32),
        ],
    )(bboxes, labels3, priors_t, plocs_t)

    lab_col = lab_out.reshape(B, N, 1)
    ceneg, scal2 = pl.pallas_call(
        _ce_kernel,
        grid=(B, NCH),
        in_specs=[
            pl.BlockSpec((1, PCH, NCLS), lambda b, c: (b, c, 0)),
            pl.BlockSpec((1, PCH, 1), lambda b, c: (b, c, 0)),
        ],
        out_specs=[
            pl.BlockSpec((1, PCH, 1), lambda b, c: (b, c, 0)),
            pl.BlockSpec((1, 1, 128), lambda b, c: (b, 0, 0)),
        ],
        out_shape=[
            jax.ShapeDtypeStruct((B, N, 1), jnp.float32),
            jax.ShapeDtypeStruct((B, 1, 128), jnp.float32),
        ],
    )(p_clss, lab_col)

    ceneg2 = ceneg.reshape(B, N)
    out = pl.pallas_call(
        _reduce_kernel,
        grid=(1,),
        in_specs=[
            pl.BlockSpec((B, N), lambda i: (0, 0)),
            pl.BlockSpec((B, 1, 128), lambda i: (0, 0, 0)),
            pl.BlockSpec((B, 1, 128), lambda i: (0, 0, 0)),
        ],
        out_specs=pl.BlockSpec((8, 128), lambda i: (0, 0)),
        out_shape=jax.ShapeDtypeStruct((8, 128), jnp.float32),
    )(ceneg2, scal1, scal2)
    return out[0, 0]


# speedup vs baseline: 12.7770x; 1.4792x over previous
import jax
import jax.numpy as jnp
from jax.experimental import pallas as pl

B, N, NOBJ, NCLS = 16, 24564, 16, 81
THR = 0.5
NEG_RATIO = 3.0


def _match_kernel(bboxes_ref, bbt_ref, labels_f_ref, priors_t_ref,
                  plocs_t_ref, label_out_ref, scal_out_ref):
    pcx = priors_t_ref[0:1, :]
    pcy = priors_t_ref[1:2, :]
    pw = priors_t_ref[2:3, :]
    ph = priors_t_ref[3:4, :]
    px0 = pcx - pw / 2.0
    py0 = pcy - ph / 2.0
    px1 = pcx + pw / 2.0
    py1 = pcy + ph / 2.0

    bb = bboxes_ref[0]
    bx0 = bb[:, 0:1]
    by0 = bb[:, 1:2]
    bx1 = bb[:, 2:3]
    by1 = bb[:, 3:4]

    iw = jnp.maximum(jnp.minimum(bx1, px1) - jnp.maximum(bx0, px0), 0.0)
    ih = jnp.maximum(jnp.minimum(by1, py1) - jnp.maximum(by0, py0), 0.0)
    inter = iw * ih
    area_a = (bx1 - bx0) * (by1 - by0)
    area_b = (px1 - px0) * (py1 - py0)
    sc = inter / (area_a + area_b - inter)

    obj_iota = jax.lax.broadcasted_iota(jnp.int32, (NOBJ, N), 0)
    lane_iota = jax.lax.broadcasted_iota(jnp.int32, (NOBJ, N), 1)

    best_iou = jnp.max(sc, axis=0, keepdims=True)
    pbb = jnp.min(jnp.where(sc == best_iou, obj_iota, NOBJ), axis=0,
                  keepdims=True)
    bbp_val = jnp.max(sc, axis=1, keepdims=True)
    bbp = jnp.min(jnp.where(sc == bbp_val, lane_iota, N), axis=1,
                  keepdims=True)

    eq = lane_iota == bbp
    fj = jnp.max(jnp.where(eq, obj_iota, -1), axis=0, keepdims=True)
    forced = fj >= 0
    pbb = jnp.where(forced, fj, pbb)
    best_iou = jnp.where(forced, 1.0, best_iou)

    onehot = (obj_iota == pbb).astype(jnp.float32)
    table = jnp.concatenate([labels_f_ref[0], bbt_ref[0]], axis=0)
    g = jax.lax.dot_general(table, onehot, (((1,), (0,)), ((), ())),
                            preferred_element_type=jnp.float32)
    gx0 = g[1:2, :]
    gy0 = g[2:3, :]
    gx1 = g[3:4, :]
    gy1 = g[4:5, :]
    lbl = jnp.where(best_iou < THR, 0, g[0:1, :].astype(jnp.int32))
    pos = lbl != 0

    cx = (gx0 + gx1) / 2.0
    cy = (gy0 + gy1) / 2.0
    w = gx1 - gx0
    h = gy1 - gy0
    tl = jnp.concatenate(
        [(cx - pcx) / (pw / 10.0),
         (cy - pcy) / (ph / 10.0),
         jnp.log(w / pw) * 5.0,
         jnp.log(h / ph) * 5.0], axis=0)

    pl4 = plocs_t_ref[0]
    loc_l1 = jnp.sum(jnp.where(pos, jnp.abs(pl4 - tl), 0.0))
    n_pos = jnp.sum(jnp.where(pos, 1.0, 0.0))

    label_out_ref[:] = lbl.reshape(1, 1, N)
    io = jax.lax.broadcasted_iota(jnp.int32, (1, 128), 1)
    scal = jnp.where(io == 0, n_pos, jnp.where(io == 1, loc_l1, 0.0))
    scal_out_ref[:] = scal.reshape(1, 1, 128)


PCH = 4096
NCH = (N + PCH - 1) // PCH


def _ce_kernel(pclss_ref, labc_ref, labr_ref, ceneg_out_ref, scal_out_ref):
    c = pl.program_id(1)
    x = pclss_ref[0]
    lblc = labc_ref[0]
    lblr = labr_ref[0]
    xm = x - x[:, 0:1]
    ex = jnp.exp(xm)
    lhs = (jax.lax.broadcasted_iota(jnp.int32, (8, NCLS), 0) == 0)
    st = jax.lax.dot_general(lhs.astype(jnp.float32), ex,
                             (((1,), (1,)), ((), ())),
                             preferred_element_type=jnp.float32)
    logs = jnp.log(st[0:1, :])
    posr = lblr != 0
    ceneg_out_ref[:] = jnp.where(posr, 0.0, logs).reshape(1, 1, PCH)

    cls_iota = jax.lax.broadcasted_iota(jnp.int32, (PCH, NCLS), 1)
    selp = jnp.logical_and(cls_iota == lblc, cls_iota > 0)
    xsum = jnp.sum(jnp.where(selp, xm, 0.0))
    conf_pos = jnp.sum(jnp.where(posr, logs, 0.0)) - xsum

    @pl.when(c == 0)
    def _():
        scal_out_ref[:] = jnp.zeros((1, 1, 128), jnp.float32)

    io = jax.lax.broadcasted_iota(jnp.int32, (1, 128), 1)
    scal_out_ref[:] = scal_out_ref[:] + jnp.where(
        io == 0, conf_pos, 0.0).reshape(1, 1, 128)


def _reduce_kernel(ceneg_ref, scal1_ref, scal2_ref, out_ref):
    v = ceneg_ref[:]
    n_pos = scal1_ref[:, 0, 0:1]
    loc_l1 = scal1_ref[:, 0, 1:2]
    conf_pos = scal2_ref[:, 0, 0:1]
    k = jnp.minimum(n_pos * NEG_RATIO, float(N))

    prefix = jnp.zeros((B, 1), jnp.int32)
    for bit in range(30, -1, -1):
        cand = prefix | (1 << bit)
        candf = jax.lax.bitcast_convert_type(cand, jnp.float32)
        cnt = jnp.sum(jnp.where(v >= candf, 1.0, 0.0), axis=1, keepdims=True)
        prefix = jnp.where(cnt >= k, cand, prefix)
    t = jax.lax.bitcast_convert_type(prefix, jnp.float32)

    gt = v > t
    cnt_gt = jnp.sum(jnp.where(gt, 1.0, 0.0), axis=1, keepdims=True)
    sum_gt = jnp.sum(jnp.where(gt, v, 0.0), axis=1, keepdims=True)
    hard = sum_gt + (k - cnt_gt) * t

    n_total = jnp.sum(n_pos)
    conf = (jnp.sum(hard) + jnp.sum(conf_pos)) / n_total
    loc = jnp.sum(loc_l1) / (n_total * 4.0)
    out_ref[:] = jnp.full((8, 128), conf + loc, jnp.float32)


@jax.jit
def kernel(p_locs, p_clss, bboxes, labels, priors):
    priors_t = priors.T
    plocs_t = p_locs.transpose(0, 2, 1)
    bboxes_t = bboxes.transpose(0, 2, 1)
    labels_f = labels.astype(jnp.float32).reshape(B, 1, NOBJ)

    lab_out, scal1 = pl.pallas_call(
        _match_kernel,
        grid=(B,),
        in_specs=[
            pl.BlockSpec((1, NOBJ, 4), lambda b: (b, 0, 0)),
            pl.BlockSpec((1, 4, NOBJ), lambda b: (b, 0, 0)),
            pl.BlockSpec((1, 1, NOBJ), lambda b: (b, 0, 0)),
            pl.BlockSpec((4, N), lambda b: (0, 0)),
            pl.BlockSpec((1, 4, N), lambda b: (b, 0, 0)),
        ],
        out_specs=[
            pl.BlockSpec((1, 1, N), lambda b: (b, 0, 0)),
            pl.BlockSpec((1, 1, 128), lambda b: (b, 0, 0)),
        ],
        out_shape=[
            jax.ShapeDtypeStruct((B, 1, N), jnp.int32),
            jax.ShapeDtypeStruct((B, 1, 128), jnp.float32),
        ],
    )(bboxes, bboxes_t, labels_f, priors_t, plocs_t)

    lab_row = jnp.pad(lab_out, ((0, 0), (0, 0), (0, NCH * PCH - N)))
    lab_col = lab_row.reshape(B, NCH * PCH, 1)
    ceneg, scal2 = pl.pallas_call(
        _ce_kernel,
        grid=(B, NCH),
        in_specs=[
            pl.BlockSpec((1, PCH, NCLS), lambda b, c: (b, c, 0)),
            pl.BlockSpec((1, PCH, 1), lambda b, c: (b, c, 0)),
            pl.BlockSpec((1, 1, PCH), lambda b, c: (b, 0, c)),
        ],
        out_specs=[
            pl.BlockSpec((1, 1, PCH), lambda b, c: (b, 0, c)),
            pl.BlockSpec((1, 1, 128), lambda b, c: (b, 0, 0)),
        ],
        out_shape=[
            jax.ShapeDtypeStruct((B, 1, N), jnp.float32),
            jax.ShapeDtypeStruct((B, 1, 128), jnp.float32),
        ],
    )(p_clss, lab_col, lab_row)

    ceneg2 = ceneg.reshape(B, N)
    out = pl.pallas_call(
        _reduce_kernel,
        grid=(1,),
        in_specs=[
            pl.BlockSpec((B, N), lambda i: (0, 0)),
            pl.BlockSpec((B, 1, 128), lambda i: (0, 0, 0)),
            pl.BlockSpec((B, 1, 128), lambda i: (0, 0, 0)),
        ],
        out_specs=pl.BlockSpec((8, 128), lambda i: (0, 0)),
        out_shape=jax.ShapeDtypeStruct((8, 128), jnp.float32),
    )(ceneg2, scal1, scal2)
    return out[0, 0]


# speedup vs baseline: 16.4142x; 1.2847x over previous
import jax
import jax.numpy as jnp
from jax.experimental import pallas as pl

B, N, NOBJ, NCLS = 16, 24564, 16, 81
THR = 0.5
NEG_RATIO = 3.0


def _match_kernel(bboxes_ref, bbt_ref, labels_f_ref, priors_t_ref,
                  plocs_t_ref, label_out_ref, scal_out_ref):
    pcx = priors_t_ref[0:1, :]
    pcy = priors_t_ref[1:2, :]
    pw = priors_t_ref[2:3, :]
    ph = priors_t_ref[3:4, :]
    px0 = pcx - pw / 2.0
    py0 = pcy - ph / 2.0
    px1 = pcx + pw / 2.0
    py1 = pcy + ph / 2.0

    bb = bboxes_ref[0]
    bx0 = bb[:, 0:1]
    by0 = bb[:, 1:2]
    bx1 = bb[:, 2:3]
    by1 = bb[:, 3:4]

    iw = jnp.maximum(jnp.minimum(bx1, px1) - jnp.maximum(bx0, px0), 0.0)
    ih = jnp.maximum(jnp.minimum(by1, py1) - jnp.maximum(by0, py0), 0.0)
    inter = iw * ih
    area_a = (bx1 - bx0) * (by1 - by0)
    area_b = (px1 - px0) * (py1 - py0)
    sc = inter / (area_a + area_b - inter)

    obj_iota = jax.lax.broadcasted_iota(jnp.int32, (NOBJ, N), 0)
    lane_iota = jax.lax.broadcasted_iota(jnp.int32, (NOBJ, N), 1)

    best_iou = jnp.max(sc, axis=0, keepdims=True)
    pbb = jnp.min(jnp.where(sc == best_iou, obj_iota, NOBJ), axis=0,
                  keepdims=True)
    bbp_val = jnp.max(sc, axis=1, keepdims=True)
    bbp = jnp.min(jnp.where(sc == bbp_val, lane_iota, N), axis=1,
                  keepdims=True)

    eq = lane_iota == bbp
    fj = jnp.max(jnp.where(eq, obj_iota, -1), axis=0, keepdims=True)
    forced = fj >= 0
    pbb = jnp.where(forced, fj, pbb)
    best_iou = jnp.where(forced, 1.0, best_iou)

    onehot = (obj_iota == pbb).astype(jnp.float32)
    table = jnp.concatenate([labels_f_ref[0], bbt_ref[0]], axis=0)
    g = jax.lax.dot_general(table, onehot, (((1,), (0,)), ((), ())),
                            preferred_element_type=jnp.float32)
    gx0 = g[1:2, :]
    gy0 = g[2:3, :]
    gx1 = g[3:4, :]
    gy1 = g[4:5, :]
    lbl = jnp.where(best_iou < THR, 0, g[0:1, :].astype(jnp.int32))
    pos = lbl != 0

    cx = (gx0 + gx1) / 2.0
    cy = (gy0 + gy1) / 2.0
    w = gx1 - gx0
    h = gy1 - gy0
    tl = jnp.concatenate(
        [(cx - pcx) / (pw / 10.0),
         (cy - pcy) / (ph / 10.0),
         jnp.log(w / pw) * 5.0,
         jnp.log(h / ph) * 5.0], axis=0)

    pl4 = plocs_t_ref[0]
    loc_l1 = jnp.sum(jnp.where(pos, jnp.abs(pl4 - tl), 0.0))
    n_pos = jnp.sum(jnp.where(pos, 1.0, 0.0))

    label_out_ref[:] = lbl.reshape(1, 1, N)
    io = jax.lax.broadcasted_iota(jnp.int32, (1, 128), 1)
    scal = jnp.where(io == 0, n_pos, jnp.where(io == 1, loc_l1, 0.0))
    scal_out_ref[:] = scal.reshape(1, 1, 128)


PCH = 4096
NCH = (N + PCH - 1) // PCH


def _ce_kernel(pclss_ref, labr_ref, ceneg_out_ref, scal_out_ref):
    c = pl.program_id(1)
    x = pclss_ref[0]
    lane = jax.lax.broadcasted_iota(jnp.int32, (1, PCH), 1) + c * PCH
    lblr = jnp.where(lane < N, labr_ref[0], 0)
    lblc = lblr.reshape(PCH, 1)
    xm = x - x[:, 0:1]
    ex = jnp.exp(xm)
    lhs = (jax.lax.broadcasted_iota(jnp.int32, (8, NCLS), 0) == 0)
    st = jax.lax.dot_general(lhs.astype(jnp.float32), ex,
                             (((1,), (1,)), ((), ())),
                             preferred_element_type=jnp.float32)
    logs = jnp.log(st[0:1, :])
    posr = lblr != 0
    ceneg_out_ref[:] = jnp.where(posr, 0.0, logs).reshape(1, 1, PCH)

    cls_iota = jax.lax.broadcasted_iota(jnp.int32, (PCH, NCLS), 1)
    selp = jnp.logical_and(cls_iota == lblc, cls_iota > 0)
    xsum = jnp.sum(jnp.where(selp, xm, 0.0))
    conf_pos = jnp.sum(jnp.where(posr, logs, 0.0)) - xsum

    @pl.when(c == 0)
    def _():
        scal_out_ref[:] = jnp.zeros((1, 1, 128), jnp.float32)

    io = jax.lax.broadcasted_iota(jnp.int32, (1, 128), 1)
    scal_out_ref[:] = scal_out_ref[:] + jnp.where(
        io == 0, conf_pos, 0.0).reshape(1, 1, 128)


def _reduce_kernel(ceneg_ref, scal1_ref, scal2_ref, out_ref):
    v = ceneg_ref[:]
    n_pos = scal1_ref[:, 0, 0:1]
    loc_l1 = scal1_ref[:, 0, 1:2]
    conf_pos = scal2_ref[:, 0, 0:1]
    k = jnp.minimum(n_pos * NEG_RATIO, float(N))

    prefix = jnp.zeros((B, 1), jnp.int32)
    for bit in range(30, -1, -1):
        cand = prefix | (1 << bit)
        candf = jax.lax.bitcast_convert_type(cand, jnp.float32)
        cnt = jnp.sum(jnp.where(v >= candf, 1.0, 0.0), axis=1, keepdims=True)
        prefix = jnp.where(cnt >= k, cand, prefix)
    t = jax.lax.bitcast_convert_type(prefix, jnp.float32)

    gt = v > t
    cnt_gt = jnp.sum(jnp.where(gt, 1.0, 0.0), axis=1, keepdims=True)
    sum_gt = jnp.sum(jnp.where(gt, v, 0.0), axis=1, keepdims=True)
    hard = sum_gt + (k - cnt_gt) * t

    n_total = jnp.sum(n_pos)
    conf = (jnp.sum(hard) + jnp.sum(conf_pos)) / n_total
    loc = jnp.sum(loc_l1) / (n_total * 4.0)
    out_ref[:] = jnp.full((8, 128), conf + loc, jnp.float32)


@jax.jit
def kernel(p_locs, p_clss, bboxes, labels, priors):
    priors_t = priors.T
    plocs_t = p_locs.transpose(0, 2, 1)
    bboxes_t = bboxes.transpose(0, 2, 1)
    labels_f = labels.astype(jnp.float32).reshape(B, 1, NOBJ)

    lab_out, scal1 = pl.pallas_call(
        _match_kernel,
        grid=(B,),
        in_specs=[
            pl.BlockSpec((1, NOBJ, 4), lambda b: (b, 0, 0)),
            pl.BlockSpec((1, 4, NOBJ), lambda b: (b, 0, 0)),
            pl.BlockSpec((1, 1, NOBJ), lambda b: (b, 0, 0)),
            pl.BlockSpec((4, N), lambda b: (0, 0)),
            pl.BlockSpec((1, 4, N), lambda b: (b, 0, 0)),
        ],
        out_specs=[
            pl.BlockSpec((1, 1, N), lambda b: (b, 0, 0)),
            pl.BlockSpec((1, 1, 128), lambda b: (b, 0, 0)),
        ],
        out_shape=[
            jax.ShapeDtypeStruct((B, 1, N), jnp.int32),
            jax.ShapeDtypeStruct((B, 1, 128), jnp.float32),
        ],
    )(bboxes, bboxes_t, labels_f, priors_t, plocs_t)

    ceneg, scal2 = pl.pallas_call(
        _ce_kernel,
        grid=(B, NCH),
        in_specs=[
            pl.BlockSpec((1, PCH, NCLS), lambda b, c: (b, c, 0)),
            pl.BlockSpec((1, 1, PCH), lambda b, c: (b, 0, c)),
        ],
        out_specs=[
            pl.BlockSpec((1, 1, PCH), lambda b, c: (b, 0, c)),
            pl.BlockSpec((1, 1, 128), lambda b, c: (b, 0, 0)),
        ],
        out_shape=[
            jax.ShapeDtypeStruct((B, 1, N), jnp.float32),
            jax.ShapeDtypeStruct((B, 1, 128), jnp.float32),
        ],
    )(p_clss, lab_out)

    ceneg2 = ceneg.reshape(B, N)
    out = pl.pallas_call(
        _reduce_kernel,
        grid=(1,),
        in_specs=[
            pl.BlockSpec((B, N), lambda i: (0, 0)),
            pl.BlockSpec((B, 1, 128), lambda i: (0, 0, 0)),
            pl.BlockSpec((B, 1, 128), lambda i: (0, 0, 0)),
        ],
        out_specs=pl.BlockSpec((8, 128), lambda i: (0, 0)),
        out_shape=jax.ShapeDtypeStruct((8, 128), jnp.float32),
    )(ceneg2, scal1, scal2)
    return out[0, 0]


# speedup vs baseline: 17.9227x; 1.0919x over previous
import jax
import jax.numpy as jnp
from jax.experimental import pallas as pl

B, N, NOBJ, NCLS = 16, 24564, 16, 81
THR = 0.5
NEG_RATIO = 3.0


def _match_kernel(bboxes_ref, bbt_ref, labels_f_ref, priors_t_ref,
                  plocs_t_ref, label_out_ref, scal_out_ref):
    pcx = priors_t_ref[0:1, :]
    pcy = priors_t_ref[1:2, :]
    pw = priors_t_ref[2:3, :]
    ph = priors_t_ref[3:4, :]
    px0 = pcx - pw / 2.0
    py0 = pcy - ph / 2.0
    px1 = pcx + pw / 2.0
    py1 = pcy + ph / 2.0

    bb = bboxes_ref[0]
    bx0 = bb[:, 0:1]
    by0 = bb[:, 1:2]
    bx1 = bb[:, 2:3]
    by1 = bb[:, 3:4]

    iw = jnp.maximum(jnp.minimum(bx1, px1) - jnp.maximum(bx0, px0), 0.0)
    ih = jnp.maximum(jnp.minimum(by1, py1) - jnp.maximum(by0, py0), 0.0)
    inter = iw * ih
    area_a = (bx1 - bx0) * (by1 - by0)
    area_b = (px1 - px0) * (py1 - py0)
    sc = inter / (area_a + area_b - inter)

    obj_iota = jax.lax.broadcasted_iota(jnp.int32, (NOBJ, N), 0)
    lane_iota = jax.lax.broadcasted_iota(jnp.int32, (NOBJ, N), 1)

    best_iou = jnp.max(sc, axis=0, keepdims=True)
    pbb = jnp.min(jnp.where(sc == best_iou, obj_iota, NOBJ), axis=0,
                  keepdims=True)
    bbp_val = jnp.max(sc, axis=1, keepdims=True)
    bbp = jnp.min(jnp.where(sc == bbp_val, lane_iota, N), axis=1,
                  keepdims=True)

    eq = lane_iota == bbp
    fj = jnp.max(jnp.where(eq, obj_iota, -1), axis=0, keepdims=True)
    forced = fj >= 0
    pbb = jnp.where(forced, fj, pbb)
    best_iou = jnp.where(forced, 1.0, best_iou)

    onehot = (obj_iota == pbb).astype(jnp.float32)
    table = jnp.concatenate([labels_f_ref[0], bbt_ref[0]], axis=0)
    g = jax.lax.dot_general(table, onehot, (((1,), (0,)), ((), ())),
                            preferred_element_type=jnp.float32)
    gx0 = g[1:2, :]
    gy0 = g[2:3, :]
    gx1 = g[3:4, :]
    gy1 = g[4:5, :]
    lbl = jnp.where(best_iou < THR, 0, g[0:1, :].astype(jnp.int32))
    pos = lbl != 0

    cx = (gx0 + gx1) / 2.0
    cy = (gy0 + gy1) / 2.0
    w = gx1 - gx0
    h = gy1 - gy0
    tl = jnp.concatenate(
        [(cx - pcx) / (pw / 10.0),
         (cy - pcy) / (ph / 10.0),
         jnp.log(w / pw) * 5.0,
         jnp.log(h / ph) * 5.0], axis=0)

    pl4 = plocs_t_ref[0]
    loc_l1 = jnp.sum(jnp.where(pos, jnp.abs(pl4 - tl), 0.0))
    n_pos = jnp.sum(jnp.where(pos, 1.0, 0.0))

    label_out_ref[:] = lbl.reshape(1, 1, N)
    io = jax.lax.broadcasted_iota(jnp.int32, (1, 128), 1)
    scal = jnp.where(io == 0, n_pos, jnp.where(io == 1, loc_l1, 0.0))
    scal_out_ref[:] = scal.reshape(1, 1, 128)


PCH = N
NCH = (N + PCH - 1) // PCH


def _ce_kernel(pclss_ref, labr_ref, ceneg_out_ref, scal_out_ref):
    c = pl.program_id(1)
    x = pclss_ref[0]
    lane = jax.lax.broadcasted_iota(jnp.int32, (1, PCH), 1) + c * PCH
    lblr = jnp.where(lane < N, labr_ref[0], 0)
    lblc = lblr.reshape(PCH, 1)
    xm = x - x[:, 0:1]
    ex = jnp.exp(xm)
    lhs = (jax.lax.broadcasted_iota(jnp.int32, (8, NCLS), 0) == 0)
    st = jax.lax.dot_general(lhs.astype(jnp.float32), ex,
                             (((1,), (1,)), ((), ())),
                             preferred_element_type=jnp.float32)
    logs = jnp.log(st[0:1, :])
    posr = lblr != 0
    ceneg_out_ref[:] = jnp.where(posr, 0.0, logs).reshape(1, 1, PCH)

    cls_iota = jax.lax.broadcasted_iota(jnp.int32, (PCH, NCLS), 1)
    selp = jnp.logical_and(cls_iota == lblc, cls_iota > 0)
    xsum = jnp.sum(jnp.where(selp, xm, 0.0))
    conf_pos = jnp.sum(jnp.where(posr, logs, 0.0)) - xsum

    @pl.when(c == 0)
    def _():
        scal_out_ref[:] = jnp.zeros((1, 1, 128), jnp.float32)

    io = jax.lax.broadcasted_iota(jnp.int32, (1, 128), 1)
    scal_out_ref[:] = scal_out_ref[:] + jnp.where(
        io == 0, conf_pos, 0.0).reshape(1, 1, 128)


def _reduce_kernel(ceneg_ref, scal1_ref, scal2_ref, out_ref):
    v = ceneg_ref[:]
    n_pos = scal1_ref[:, 0, 0:1]
    loc_l1 = scal1_ref[:, 0, 1:2]
    conf_pos = scal2_ref[:, 0, 0:1]
    k = jnp.minimum(n_pos * NEG_RATIO, float(N))

    prefix = jnp.zeros((B, 1), jnp.int32)
    for bit in range(30, -1, -1):
        cand = prefix | (1 << bit)
        candf = jax.lax.bitcast_convert_type(cand, jnp.float32)
        cnt = jnp.sum(jnp.where(v >= candf, 1.0, 0.0), axis=1, keepdims=True)
        prefix = jnp.where(cnt >= k, cand, prefix)
    t = jax.lax.bitcast_convert_type(prefix, jnp.float32)

    gt = v > t
    cnt_gt = jnp.sum(jnp.where(gt, 1.0, 0.0), axis=1, keepdims=True)
    sum_gt = jnp.sum(jnp.where(gt, v, 0.0), axis=1, keepdims=True)
    hard = sum_gt + (k - cnt_gt) * t

    n_total = jnp.sum(n_pos)
    conf = (jnp.sum(hard) + jnp.sum(conf_pos)) / n_total
    loc = jnp.sum(loc_l1) / (n_total * 4.0)
    out_ref[:] = jnp.full((8, 128), conf + loc, jnp.float32)


@jax.jit
def kernel(p_locs, p_clss, bboxes, labels, priors):
    priors_t = priors.T
    plocs_t = p_locs.transpose(0, 2, 1)
    bboxes_t = bboxes.transpose(0, 2, 1)
    labels_f = labels.astype(jnp.float32).reshape(B, 1, NOBJ)

    lab_out, scal1 = pl.pallas_call(
        _match_kernel,
        grid=(B,),
        in_specs=[
            pl.BlockSpec((1, NOBJ, 4), lambda b: (b, 0, 0)),
            pl.BlockSpec((1, 4, NOBJ), lambda b: (b, 0, 0)),
            pl.BlockSpec((1, 1, NOBJ), lambda b: (b, 0, 0)),
            pl.BlockSpec((4, N), lambda b: (0, 0)),
            pl.BlockSpec((1, 4, N), lambda b: (b, 0, 0)),
        ],
        out_specs=[
            pl.BlockSpec((1, 1, N), lambda b: (b, 0, 0)),
            pl.BlockSpec((1, 1, 128), lambda b: (b, 0, 0)),
        ],
        out_shape=[
            jax.ShapeDtypeStruct((B, 1, N), jnp.int32),
            jax.ShapeDtypeStruct((B, 1, 128), jnp.float32),
        ],
    )(bboxes, bboxes_t, labels_f, priors_t, plocs_t)

    ceneg, scal2 = pl.pallas_call(
        _ce_kernel,
        grid=(B, NCH),
        in_specs=[
            pl.BlockSpec((1, PCH, NCLS), lambda b, c: (b, c, 0)),
            pl.BlockSpec((1, 1, PCH), lambda b, c: (b, 0, c)),
        ],
        out_specs=[
            pl.BlockSpec((1, 1, PCH), lambda b, c: (b, 0, c)),
            pl.BlockSpec((1, 1, 128), lambda b, c: (b, 0, 0)),
        ],
        out_shape=[
            jax.ShapeDtypeStruct((B, 1, N), jnp.float32),
            jax.ShapeDtypeStruct((B, 1, 128), jnp.float32),
        ],
    )(p_clss, lab_out)

    ceneg2 = ceneg.reshape(B, N)
    out = pl.pallas_call(
        _reduce_kernel,
        grid=(1,),
        in_specs=[
            pl.BlockSpec((B, N), lambda i: (0, 0)),
            pl.BlockSpec((B, 1, 128), lambda i: (0, 0, 0)),
            pl.BlockSpec((B, 1, 128), lambda i: (0, 0, 0)),
        ],
        out_specs=pl.BlockSpec((8, 128), lambda i: (0, 0)),
        out_shape=jax.ShapeDtypeStruct((8, 128), jnp.float32),
    )(ceneg2, scal1, scal2)
    return out[0, 0]
